# Initial kernel scaffold; baseline (speedup 1.0000x reference)
import jax
import jax.numpy as jnp
from jax.experimental import pallas as pl


def kernel(stim, sa_rf, sa_cn_in_rf, sa_cn_pn_rf, sa_intopn_rf, ra_rf, ra_cn_in_rf, ra_cn_pn_rf, ra_intopn_rf, cn_in_sa_rf, cn_pn_sa_rf, cn_in_ra_rf, cn_pn_ra_rf, cn_intopn_rf, sa_cn_SD, sa_intopn_DN, ra_cn_SD, ra_intopn_DN, cn_sa_SD, cn_ra_SD, cn_intopn_DN):
    raise NotImplementedError("write your pallas kernel here")


# speedup vs baseline: 1851.0204x; 1851.0204x over previous
import jax
import jax.numpy as jnp
import numpy as np
from jax.experimental import pallas as pl
from jax.experimental.pallas import tpu as pltpu

N_SA = 512; N_RA = 512; N_IN = 128; N_PN = 128; N_CN = 64
T = 120; MAX_DELAY = 10; BUF = MAX_DELAY + 1
TS = T - 1
PAD = 16
NBLK = (TS + 7) // 8
HROWS = PAD + 8 * NBLK + 8
CROWS = 8 * NBLK
V_THRES = 30.0


def _decay_filter(tau: float) -> np.ndarray:
    e = np.float32(np.exp(np.float32(-1.0) / np.float32(tau)))
    k = np.arange(TS)
    pw = np.power(np.float64(e), k).astype(np.float32)
    L = np.zeros((TS, TS), np.float32)
    for r in range(TS):
        L[r, : r + 1] = pw[: r + 1][::-1]
    return L


def _a_schedule() -> np.ndarray:
    a = np.float32(0.02)
    out = np.zeros((CROWS, 1), np.float32)
    for r in range(TS):
        out[r, 0] = a
        a = np.float32(a * np.float32(1.01))
    return out


def _snn_body(stim_ref, e5_ref, e10_ref, a_sa_ref,
              sa_rfT_ref, ra_rfT_ref,
              wt_sa_ref, wt_ra_ref, wt_saio_ref, wt_raio_ref,
              wt_cnsa_ref, wt_cnra_ref, wt_cnio_ref,
              spk_sa0, spk_sa1, spk_sa2, spk_ra0, spk_ra1, spk_ra2,
              spk_cn0, spk_cn1,
              cur_a, cur_b, cur_c, cur_d, cur_e, cur_f):
    f32 = jnp.float32

    for ref, n in ((spk_sa0, N_SA), (spk_sa1, N_IN), (spk_sa2, N_PN),
                   (spk_ra0, N_RA), (spk_ra1, N_IN), (spk_ra2, N_PN),
                   (spk_cn0, N_CN), (spk_cn1, N_CN)):
        ref[0:PAD, :] = jnp.zeros((PAD, n), f32)

    def shift_matmul(spk_ref, wt_ref, post):
        acc = jnp.zeros((TS, post), f32)
        for d in range(1, MAX_DELAY + 1):
            sl = spk_ref[PAD - d:PAD - d + TS, :]
            acc = acc + jnp.dot(sl, wt_ref[d - 1], preferred_element_type=f32,
                      precision=jax.lax.Precision.HIGHEST)
        return acc

    def scan(layers):
        inits = []
        for (_, _, n, _, _) in layers:
            inits.append(jnp.full((1, n), -65.0, f32))
            inits.append(jnp.full((1, n), -13.0, f32))

        def body(k, carry):
            r8 = k * 8
            a_blk = a_sa_ref[pl.ds(r8, 8), :]
            out = []
            for i, (cur_ref, spk_ref, n, d_jump, a_mode) in enumerate(layers):
                v, u = carry[2 * i], carry[2 * i + 1]
                blk = cur_ref[pl.ds(r8, 8), 0:n]
                rows = []
                for j in range(8):
                    I = blk[j:j + 1, :]
                    v = v + 0.5 * (0.04 * v * v + 5.0 * v + 140.0 - u + I)
                    v = v + 0.5 * (0.04 * v * v + 5.0 * v + 140.0 - u + I)
                    a = a_blk[j:j + 1, :] if a_mode == 'sa0' else a_mode
                    u = u + a * (0.2 * v - u)
                    spk = (v >= V_THRES).astype(f32)
                    v = jnp.where(spk > 0, -65.0, v)
                    u = jnp.where(spk > 0, u + d_jump, u)
                    rows.append(spk)
                spk_ref[pl.ds(PAD + r8, 8), :] = jnp.concatenate(rows, axis=0)
                out.append(v)
                out.append(u)
            return tuple(out)

        jax.lax.fori_loop(0, NBLK, body, tuple(inits))

    stim_sa = stim_ref[1:T, :]
    stim_ra = jnp.abs(stim_ref[1:T, :] - stim_ref[0:T - 1, :]) * 5.0
    e5 = e5_ref[...]
    e10 = e10_ref[...]
    psp_sa = jnp.dot(e5, stim_sa, preferred_element_type=f32,
                      precision=jax.lax.Precision.HIGHEST)
    psp_ra = jnp.dot(e5, stim_ra, preferred_element_type=f32,
                      precision=jax.lax.Precision.HIGHEST)
    cur_a[0:TS, :] = jnp.dot(psp_sa, sa_rfT_ref[...], preferred_element_type=f32,
                      precision=jax.lax.Precision.HIGHEST)
    cur_b[0:TS, :] = jnp.dot(psp_ra, ra_rfT_ref[...], preferred_element_type=f32,
                      precision=jax.lax.Precision.HIGHEST)
    scan([(cur_a, spk_sa0, N_SA, 8.0, 'sa0'),
          (cur_b, spk_ra0, N_RA, 2.0, 0.02)])

    c_sa = shift_matmul(spk_sa0, wt_sa_ref, 2 * N_IN)
    c_ra = shift_matmul(spk_ra0, wt_ra_ref, 2 * N_IN)
    cur_c[0:TS, :] = jnp.dot(e5, c_sa[:, 0:N_IN], preferred_element_type=f32,
                      precision=jax.lax.Precision.HIGHEST)
    cur_d[0:TS, :] = jnp.dot(e5, c_ra[:, 0:N_IN], preferred_element_type=f32,
                      precision=jax.lax.Precision.HIGHEST)
    cur_e[0:TS, :] = jnp.dot(e5, c_sa[:, N_IN:2 * N_IN],
                             preferred_element_type=f32,
                      precision=jax.lax.Precision.HIGHEST)
    cur_f[0:TS, :] = jnp.dot(e5, c_ra[:, N_IN:2 * N_IN],
                             preferred_element_type=f32,
                      precision=jax.lax.Precision.HIGHEST)
    scan([(cur_c, spk_sa1, N_IN, 6.0, 0.1),
          (cur_d, spk_ra1, N_IN, 2.0, 0.1)])

    c_saio = shift_matmul(spk_sa1, wt_saio_ref, N_PN)
    c_raio = shift_matmul(spk_ra1, wt_raio_ref, N_PN)
    cur_a[0:TS, 0:N_PN] = cur_e[0:TS, :] - jnp.dot(
        e10, c_saio, preferred_element_type=f32,
                      precision=jax.lax.Precision.HIGHEST)
    cur_b[0:TS, 0:N_PN] = cur_f[0:TS, :] - jnp.dot(
        e10, c_raio, preferred_element_type=f32,
                      precision=jax.lax.Precision.HIGHEST)
    scan([(cur_a, spk_sa2, N_PN, 6.0, 0.1),
          (cur_b, spk_ra2, N_PN, 2.0, 0.1)])

    c_cnsa = shift_matmul(spk_sa2, wt_cnsa_ref, 2 * N_CN)
    c_cnra = shift_matmul(spk_ra2, wt_cnra_ref, 2 * N_CN)
    cur_c[0:TS, 0:N_CN] = jnp.dot(
        e5, c_cnsa[:, 0:N_CN] + c_cnra[:, 0:N_CN], preferred_element_type=f32,
                      precision=jax.lax.Precision.HIGHEST)
    scan([(cur_c, spk_cn0, N_CN, 8.0, 0.02)])

    c_cnio = shift_matmul(spk_cn0, wt_cnio_ref, N_CN)
    cur_d[0:TS, 0:N_CN] = (
        2.0 * jnp.dot(e5, c_cnsa[:, N_CN:2 * N_CN] + c_cnra[:, N_CN:2 * N_CN],
                      preferred_element_type=f32,
                      precision=jax.lax.Precision.HIGHEST)
        - jnp.dot(e10, c_cnio, preferred_element_type=f32,
                      precision=jax.lax.Precision.HIGHEST))
    scan([(cur_d, spk_cn1, N_CN, 8.0, 0.02)])


def kernel(stim, sa_rf, sa_cn_in_rf, sa_cn_pn_rf, sa_intopn_rf,
           ra_rf, ra_cn_in_rf, ra_cn_pn_rf, ra_intopn_rf,
           cn_in_sa_rf, cn_pn_sa_rf, cn_in_ra_rf, cn_pn_ra_rf, cn_intopn_rf,
           sa_cn_SD, sa_intopn_DN, ra_cn_SD, ra_intopn_DN,
           cn_sa_SD, cn_ra_SD, cn_intopn_DN):
    f32 = jnp.float32

    stim_t = jnp.transpose(stim[0], (1, 0))

    e5 = jnp.asarray(_decay_filter(5.0))
    e10 = jnp.asarray(_decay_filter(10.0))
    a_sa = jnp.asarray(_a_schedule())

    wt_sa = _mask_stack(sa_cn_in_rf, sa_cn_pn_rf, sd=sa_cn_SD)
    wt_ra = _mask_stack(ra_cn_in_rf, ra_cn_pn_rf, sd=ra_cn_SD)
    wt_saio = _mask_stack(sa_intopn_rf, sd=sa_intopn_DN)
    wt_raio = _mask_stack(ra_intopn_rf, sd=ra_intopn_DN)
    wt_cnsa = _mask_stack(cn_in_sa_rf, cn_pn_sa_rf, sd=cn_sa_SD)
    wt_cnra = _mask_stack(cn_in_ra_rf, cn_pn_ra_rf, sd=cn_ra_SD)
    wt_cnio = _mask_stack(cn_intopn_rf, sd=cn_intopn_DN)

    out_shapes = [
        jax.ShapeDtypeStruct((HROWS, N_SA), f32),
        jax.ShapeDtypeStruct((HROWS, N_IN), f32),
        jax.ShapeDtypeStruct((HROWS, N_PN), f32),
        jax.ShapeDtypeStruct((HROWS, N_RA), f32),
        jax.ShapeDtypeStruct((HROWS, N_IN), f32),
        jax.ShapeDtypeStruct((HROWS, N_PN), f32),
        jax.ShapeDtypeStruct((HROWS, N_CN), f32),
        jax.ShapeDtypeStruct((HROWS, N_CN), f32),
    ]
    scratch = [
        pltpu.VMEM((CROWS, N_SA), f32),
        pltpu.VMEM((CROWS, N_RA), f32),
        pltpu.VMEM((CROWS, N_IN), f32),
        pltpu.VMEM((CROWS, N_IN), f32),
        pltpu.VMEM((CROWS, N_PN), f32),
        pltpu.VMEM((CROWS, N_PN), f32),
    ]

    outs = pl.pallas_call(
        _snn_body,
        out_shape=out_shapes,
        scratch_shapes=scratch,
    )(stim_t, e5, e10, a_sa,
      jnp.transpose(sa_rf, (1, 0)), jnp.transpose(ra_rf, (1, 0)),
      wt_sa, wt_ra, wt_saio, wt_raio, wt_cnsa, wt_cnra, wt_cnio)

    rows = outs
    return jnp.concatenate(
        [jnp.transpose(r[PAD:PAD + TS, :], (1, 0)) for r in rows], axis=0)


def _mask_stack(*w_list, sd):
    outs = []
    sdi = sd.astype(jnp.int32)
    for d in range(1, MAX_DELAY + 1):
        mask = (sdi == d).astype(jnp.float32)
        cols = [jnp.transpose(w * mask, (1, 0)) for w in w_list]
        outs.append(jnp.concatenate(cols, axis=1))
    return jnp.stack(outs, axis=0)


# speedup vs baseline: 5495.4658x; 2.9689x over previous
import jax
import jax.numpy as jnp
import numpy as np
from jax.experimental import pallas as pl
from jax.experimental.pallas import tpu as pltpu

N_SA = 512; N_RA = 512; N_IN = 128; N_PN = 128; N_CN = 64
T = 120; MAX_DELAY = 10; BUF = MAX_DELAY + 1
TS = T - 1
PAD = 16
NBLK = (TS + 7) // 8
HROWS = PAD + 8 * NBLK + 8
CROWS = 8 * NBLK
V_THRES = 30.0

_HI = jax.lax.Precision.HIGHEST
_DN_RR = (((1,), (1,)), ((), ()))


def _decay_filter(tau: float) -> np.ndarray:
    e = np.float32(np.exp(np.float32(-1.0) / np.float32(tau)))
    k = np.arange(TS)
    pw = np.power(np.float64(e), k).astype(np.float32)
    L = np.zeros((TS, TS), np.float32)
    for r in range(TS):
        L[r, : r + 1] = pw[: r + 1][::-1]
    return L


def _a_schedule() -> np.ndarray:
    a = np.float32(0.02)
    out = np.zeros((CROWS, 1), np.float32)
    for r in range(TS):
        out[r, 0] = a
        a = np.float32(a * np.float32(1.01))
    return out


def _snn_body(stim_ref, e5_ref, e10_ref, a_sa_ref,
              sa_rf_ref, ra_rf_ref,
              sa_cn_in_ref, sa_cn_pn_ref, sa_io_ref, ra_cn_in_ref,
              ra_cn_pn_ref, ra_io_ref, cn_in_sa_ref, cn_pn_sa_ref,
              cn_in_ra_ref, cn_pn_ra_ref, cn_io_ref,
              sa_cn_sd_ref, sa_io_sd_ref, ra_cn_sd_ref, ra_io_sd_ref,
              cn_sa_sd_ref, cn_ra_sd_ref, cn_io_sd_ref,
              spk_sa0, spk_sa1, spk_sa2, spk_ra0, spk_ra1, spk_ra2,
              spk_cn0, spk_cn1,
              cur_a, cur_b, cur_c, cur_d, cur_e, cur_f):
    f32 = jnp.float32

    for ref, n in ((spk_sa0, N_SA), (spk_sa1, N_IN), (spk_sa2, N_PN),
                   (spk_ra0, N_RA), (spk_ra1, N_IN), (spk_ra2, N_PN),
                   (spk_cn0, N_CN), (spk_cn1, N_CN)):
        ref[0:PAD, :] = jnp.zeros((PAD, n), f32)

    def shift_matmul(spk_ref, w_refs, sd_ref, post_each):
        sd = sd_ref[...]
        accs = [jnp.zeros((TS, post_each), f32) for _ in w_refs]
        for d in range(1, MAX_DELAY + 1):
            mask = (sd == d).astype(f32)
            sl = spk_ref[PAD - d:PAD - d + TS, :]
            for i, w_ref in enumerate(w_refs):
                wd = w_ref[...] * mask
                accs[i] = accs[i] + jax.lax.dot_general(
                    sl, wd, _DN_RR, precision=_HI,
                    preferred_element_type=f32)
        return accs

    def scan(layers):
        inits = []
        for (_, _, n, _, _) in layers:
            inits.append(jnp.full((1, n), -65.0, f32))
            inits.append(jnp.full((1, n), -13.0, f32))

        def body(k, carry):
            r8 = k * 8
            a_blk = a_sa_ref[pl.ds(r8, 8), :]
            out = []
            for i, (cur_ref, spk_ref, n, d_jump, a_mode) in enumerate(layers):
                v, u = carry[2 * i], carry[2 * i + 1]
                blk = cur_ref[pl.ds(r8, 8), 0:n]
                rows = []
                for j in range(8):
                    I = blk[j:j + 1, :]
                    v = v + 0.5 * (0.04 * v * v + 5.0 * v + 140.0 - u + I)
                    v = v + 0.5 * (0.04 * v * v + 5.0 * v + 140.0 - u + I)
                    a = a_blk[j:j + 1, :] if a_mode == 'sa0' else a_mode
                    u = u + a * (0.2 * v - u)
                    spk = (v >= V_THRES).astype(f32)
                    v = jnp.where(spk > 0, -65.0, v)
                    u = jnp.where(spk > 0, u + d_jump, u)
                    rows.append(spk)
                spk_ref[pl.ds(PAD + r8, 8), :] = jnp.concatenate(rows, axis=0)
                out.append(v)
                out.append(u)
            return tuple(out)

        jax.lax.fori_loop(0, NBLK, body, tuple(inits))

    stim_sa = stim_ref[1:T, :]
    stim_ra = jnp.abs(stim_ref[1:T, :] - stim_ref[0:T - 1, :]) * 5.0
    e5 = e5_ref[...]
    e10 = e10_ref[...]
    psp_sa = jnp.dot(e5, stim_sa, precision=_HI, preferred_element_type=f32)
    psp_ra = jnp.dot(e5, stim_ra, precision=_HI, preferred_element_type=f32)
    cur_a[0:TS, :] = jax.lax.dot_general(
        psp_sa, sa_rf_ref[...], _DN_RR, precision=_HI,
        preferred_element_type=f32)
    cur_b[0:TS, :] = jax.lax.dot_general(
        psp_ra, ra_rf_ref[...], _DN_RR, precision=_HI,
        preferred_element_type=f32)
    scan([(cur_a, spk_sa0, N_SA, 8.0, 'sa0'),
          (cur_b, spk_ra0, N_RA, 2.0, 0.02)])

    c_sa_in, c_sa_pn = shift_matmul(spk_sa0, (sa_cn_in_ref, sa_cn_pn_ref),
                                    sa_cn_sd_ref, N_IN)
    c_ra_in, c_ra_pn = shift_matmul(spk_ra0, (ra_cn_in_ref, ra_cn_pn_ref),
                                    ra_cn_sd_ref, N_IN)
    cur_c[0:TS, :] = jnp.dot(e5, c_sa_in, precision=_HI,
                             preferred_element_type=f32)
    cur_d[0:TS, :] = jnp.dot(e5, c_ra_in, precision=_HI,
                             preferred_element_type=f32)
    cur_e[0:TS, :] = jnp.dot(e5, c_sa_pn, precision=_HI,
                             preferred_element_type=f32)
    cur_f[0:TS, :] = jnp.dot(e5, c_ra_pn, precision=_HI,
                             preferred_element_type=f32)
    scan([(cur_c, spk_sa1, N_IN, 6.0, 0.1),
          (cur_d, spk_ra1, N_IN, 2.0, 0.1)])

    (c_saio,) = shift_matmul(spk_sa1, (sa_io_ref,), sa_io_sd_ref, N_PN)
    (c_raio,) = shift_matmul(spk_ra1, (ra_io_ref,), ra_io_sd_ref, N_PN)
    cur_a[0:TS, 0:N_PN] = cur_e[0:TS, :] - jnp.dot(
        e10, c_saio, precision=_HI, preferred_element_type=f32)
    cur_b[0:TS, 0:N_PN] = cur_f[0:TS, :] - jnp.dot(
        e10, c_raio, precision=_HI, preferred_element_type=f32)
    scan([(cur_a, spk_sa2, N_PN, 6.0, 0.1),
          (cur_b, spk_ra2, N_PN, 2.0, 0.1)])

    c_cnsa_in, c_cnsa_pn = shift_matmul(spk_sa2, (cn_in_sa_ref, cn_pn_sa_ref),
                                        cn_sa_sd_ref, N_CN)
    c_cnra_in, c_cnra_pn = shift_matmul(spk_ra2, (cn_in_ra_ref, cn_pn_ra_ref),
                                        cn_ra_sd_ref, N_CN)
    cur_c[0:TS, 0:N_CN] = jnp.dot(e5, c_cnsa_in + c_cnra_in, precision=_HI,
                                  preferred_element_type=f32)
    scan([(cur_c, spk_cn0, N_CN, 8.0, 0.02)])

    (c_cnio,) = shift_matmul(spk_cn0, (cn_io_ref,), cn_io_sd_ref, N_CN)
    cur_d[0:TS, 0:N_CN] = (
        2.0 * jnp.dot(e5, c_cnsa_pn + c_cnra_pn, precision=_HI,
                      preferred_element_type=f32)
        - jnp.dot(e10, c_cnio, precision=_HI, preferred_element_type=f32))
    scan([(cur_d, spk_cn1, N_CN, 8.0, 0.02)])


def kernel(stim, sa_rf, sa_cn_in_rf, sa_cn_pn_rf, sa_intopn_rf,
           ra_rf, ra_cn_in_rf, ra_cn_pn_rf, ra_intopn_rf,
           cn_in_sa_rf, cn_pn_sa_rf, cn_in_ra_rf, cn_pn_ra_rf, cn_intopn_rf,
           sa_cn_SD, sa_intopn_DN, ra_cn_SD, ra_intopn_DN,
           cn_sa_SD, cn_ra_SD, cn_intopn_DN):
    f32 = jnp.float32

    stim_t = jnp.transpose(stim[0], (1, 0))

    e5 = jnp.asarray(_decay_filter(5.0))
    e10 = jnp.asarray(_decay_filter(10.0))
    a_sa = jnp.asarray(_a_schedule())

    out_shapes = [
        jax.ShapeDtypeStruct((HROWS, N_SA), f32),
        jax.ShapeDtypeStruct((HROWS, N_IN), f32),
        jax.ShapeDtypeStruct((HROWS, N_PN), f32),
        jax.ShapeDtypeStruct((HROWS, N_RA), f32),
        jax.ShapeDtypeStruct((HROWS, N_IN), f32),
        jax.ShapeDtypeStruct((HROWS, N_PN), f32),
        jax.ShapeDtypeStruct((HROWS, N_CN), f32),
        jax.ShapeDtypeStruct((HROWS, N_CN), f32),
    ]
    scratch = [
        pltpu.VMEM((CROWS, N_SA), f32),
        pltpu.VMEM((CROWS, N_RA), f32),
        pltpu.VMEM((CROWS, N_IN), f32),
        pltpu.VMEM((CROWS, N_IN), f32),
        pltpu.VMEM((CROWS, N_PN), f32),
        pltpu.VMEM((CROWS, N_PN), f32),
    ]

    outs = pl.pallas_call(
        _snn_body,
        out_shape=out_shapes,
        scratch_shapes=scratch,
    )(stim_t, e5, e10, a_sa,
      sa_rf, ra_rf,
      sa_cn_in_rf, sa_cn_pn_rf, sa_intopn_rf, ra_cn_in_rf,
      ra_cn_pn_rf, ra_intopn_rf, cn_in_sa_rf, cn_pn_sa_rf,
      cn_in_ra_rf, cn_pn_ra_rf, cn_intopn_rf,
      sa_cn_SD.astype(jnp.int32), sa_intopn_DN.astype(jnp.int32),
      ra_cn_SD.astype(jnp.int32), ra_intopn_DN.astype(jnp.int32),
      cn_sa_SD.astype(jnp.int32), cn_ra_SD.astype(jnp.int32),
      cn_intopn_DN.astype(jnp.int32))

    return jnp.concatenate(
        [jnp.transpose(r[PAD:PAD + TS, :], (1, 0)) for r in outs], axis=0)
